# baseline (speedup 1.0000x reference)
import functools

import jax
import jax.numpy as jnp
from jax import lax
from jax.experimental import pallas as pl
from jax.experimental.pallas import tpu as pltpu
from jax.experimental.pallas import tpu_sc as plsc

D = 1024
KGRP = 4
N_TOKENS = 8192
NW = 32
CHUNK = 64


def _sc_gather(table, idx, n_rows):
    rows_per_w = n_rows // NW
    nchunk = rows_per_w // CHUNK
    mesh = plsc.VectorSubcoreMesh(core_axis_name="c", subcore_axis_name="s")

    @functools.partial(
        pl.kernel,
        mesh=mesh,
        out_type=jax.ShapeDtypeStruct((n_rows, D), jnp.float32),
        scratch_types=[
            pltpu.VMEM((CHUNK,), jnp.int32),
            pltpu.VMEM((CHUNK, D), jnp.float32),
            pltpu.SemaphoreType.DMA,
        ],
    )
    def gather_kernel(table_hbm, idx_hbm, out_hbm, idx_v, rows_v, sem):
        wid = lax.axis_index("s") * 2 + lax.axis_index("c")
        base = wid * rows_per_w

        def body(i, carry):
            rb = base + i * CHUNK
            pltpu.sync_copy(idx_hbm.at[pl.ds(rb, CHUNK)], idx_v)
            pltpu.async_copy(table_hbm.at[idx_v], rows_v, sem).wait()
            pltpu.sync_copy(rows_v, out_hbm.at[pl.ds(rb, CHUNK)])
            return carry

        lax.fori_loop(0, nchunk, body, 0)

    return gather_kernel(table, idx)


_TM = 512


def _tc_matmul(planes, w, b2d, n_tok):
    def body(a_ref, w_ref, b_ref, o_ref):
        acc = b_ref[...].astype(jnp.float32)
        acc = jnp.broadcast_to(acc, (_TM, D))
        for k in range(KGRP):
            a = a_ref[k].astype(jnp.bfloat16)
            wk = w_ref[:, k * D:(k + 1) * D].astype(jnp.bfloat16)
            acc = acc + lax.dot_general(
                a, wk, (((1,), (1,)), ((), ())),
                preferred_element_type=jnp.float32,
            )
        o_ref[...] = acc

    return pl.pallas_call(
        body,
        grid=(n_tok // _TM,),
        in_specs=[
            pl.BlockSpec((KGRP, _TM, D), lambda i: (0, i, 0)),
            pl.BlockSpec((D, KGRP * D), lambda i: (0, 0)),
            pl.BlockSpec((1, D), lambda i: (0, 0)),
        ],
        out_specs=pl.BlockSpec((_TM, D), lambda i: (i, 0)),
        out_shape=jax.ShapeDtypeStruct((n_tok, D), jnp.float32),
    )(planes, w, b2d)


def kernel(x, emb_table, proj_w, proj_b):
    B, L, K = x.shape
    b2d = proj_b.reshape(1, D)
    outs = []
    for b in range(B):
        idx_b = x[b].T.reshape(-1).astype(jnp.int32)
        flat_b = _sc_gather(emb_table, idx_b, K * L)
        planes_b = flat_b.reshape(KGRP, L, D)
        outs.append(_tc_matmul(planes_b, proj_w, b2d, L))
    return jnp.stack(outs)


# speedup vs baseline: 1.1117x; 1.1117x over previous
import functools

import jax
import jax.numpy as jnp
from jax import lax
from jax.experimental import pallas as pl
from jax.experimental.pallas import tpu as pltpu
from jax.experimental.pallas import tpu_sc as plsc

D = 1024
KGRP = 4
N_TOKENS = 8192
NW = 32
CHUNK = 32


def _sc_gather(table, idx, n_rows):
    rows_per_w = n_rows // NW
    nchunk = rows_per_w // CHUNK
    npair = nchunk // 2
    mesh = plsc.VectorSubcoreMesh(core_axis_name="c", subcore_axis_name="s")

    @functools.partial(
        pl.kernel,
        mesh=mesh,
        out_type=jax.ShapeDtypeStruct((n_rows, D), jnp.float32),
        scratch_types=[
            pltpu.VMEM((rows_per_w,), jnp.int32),
            pltpu.VMEM((CHUNK, D), jnp.float32),
            pltpu.VMEM((CHUNK, D), jnp.float32),
            pltpu.SemaphoreType.DMA,
            pltpu.SemaphoreType.DMA,
            pltpu.SemaphoreType.DMA,
            pltpu.SemaphoreType.DMA,
        ],
    )
    def gather_kernel(table_hbm, idx_hbm, out_hbm, idx_v, rows0, rows1,
                      gs0, gs1, os0, os1):
        wid = lax.axis_index("s") * 2 + lax.axis_index("c")
        base = wid * rows_per_w
        pltpu.sync_copy(idx_hbm.at[pl.ds(base, rows_per_w)], idx_v)
        bufs = (rows0, rows1)
        gss = (gs0, gs1)
        oss = (os0, os1)

        def g_args(c, b):
            return (table_hbm.at[idx_v.at[pl.ds(c * CHUNK, CHUNK)]],
                    bufs[b], gss[b])

        def w_args(c, b):
            return (bufs[b], out_hbm.at[pl.ds(base + c * CHUNK, CHUNK)],
                    oss[b])

        pltpu.async_copy(*g_args(0, 0))
        pltpu.async_copy(*g_args(1, 1))

        def body(j, carry):
            e = 2 * j
            o = e + 1
            pltpu.make_async_copy(*g_args(e, 0)).wait()
            pltpu.async_copy(*w_args(e, 0))
            pltpu.make_async_copy(*g_args(o, 1)).wait()
            pltpu.async_copy(*w_args(o, 1))
            pltpu.make_async_copy(*w_args(e, 0)).wait()
            pltpu.async_copy(*g_args(e + 2, 0))
            pltpu.make_async_copy(*w_args(o, 1)).wait()
            pltpu.async_copy(*g_args(o + 2, 1))
            return carry

        lax.fori_loop(0, npair - 1, body, 0)

        e = nchunk - 2
        o = nchunk - 1
        pltpu.make_async_copy(*g_args(e, 0)).wait()
        pltpu.async_copy(*w_args(e, 0))
        pltpu.make_async_copy(*g_args(o, 1)).wait()
        pltpu.async_copy(*w_args(o, 1))
        pltpu.make_async_copy(*w_args(e, 0)).wait()
        pltpu.make_async_copy(*w_args(o, 1)).wait()

    return gather_kernel(table, idx)


_TM = 512


def _tc_matmul(planes, w, b2d, n_tok):
    def body(a_ref, w_ref, b_ref, o_ref):
        acc = b_ref[...].astype(jnp.float32)
        acc = jnp.broadcast_to(acc, (_TM, D))
        for k in range(KGRP):
            a = a_ref[k].astype(jnp.bfloat16)
            wk = w_ref[:, k * D:(k + 1) * D].astype(jnp.bfloat16)
            acc = acc + lax.dot_general(
                a, wk, (((1,), (1,)), ((), ())),
                preferred_element_type=jnp.float32,
            )
        o_ref[...] = acc

    return pl.pallas_call(
        body,
        grid=(n_tok // _TM,),
        in_specs=[
            pl.BlockSpec((KGRP, _TM, D), lambda i: (0, i, 0)),
            pl.BlockSpec((D, KGRP * D), lambda i: (0, 0)),
            pl.BlockSpec((1, D), lambda i: (0, 0)),
        ],
        out_specs=pl.BlockSpec((_TM, D), lambda i: (i, 0)),
        out_shape=jax.ShapeDtypeStruct((n_tok, D), jnp.float32),
    )(planes, w, b2d)


def kernel(x, emb_table, proj_w, proj_b):
    B, L, K = x.shape
    idx = x.reshape(-1, K).T.reshape(-1).astype(jnp.int32)
    flat = _sc_gather(emb_table, idx, N_TOKENS * KGRP)
    planes = flat.reshape(KGRP, N_TOKENS, D)
    out = _tc_matmul(planes, proj_w, proj_b.reshape(1, D), N_TOKENS)
    return out.reshape(B, L, D)
